# baseline (speedup 1.0000x reference)
import functools

import jax
import jax.numpy as jnp
from jax import lax
from jax.experimental import pallas as pl
from jax.experimental.pallas import tpu as pltpu
from jax.experimental.pallas import tpu_sc as plsc

ITEM_NUM = 1000000
EMB = 64
HID = 128
BATCH = 16384

_NC = 2
_NS = 16
_NW = _NC * _NS

_CHUNK = 512
_NFULL = 1953
_TAIL_BASE = _NFULL * _CHUNK
_TAIL = ITEM_NUM - _TAIL_BASE
_K = 62
_DUMP = BATCH

_sc_mesh = plsc.VectorSubcoreMesh(core_axis_name="c", subcore_axis_name="s")


@functools.partial(
    pl.kernel,
    mesh=_sc_mesh,
    out_type=jax.ShapeDtypeStruct((BATCH + 16 * _NW, EMB), jnp.float32),
    scratch_types=[
        pltpu.VMEM((BATCH + 16,), jnp.int32),
        pltpu.VMEM((BATCH + 16,), jnp.int32),
        pltpu.VMEM((BATCH + 16,), jnp.int32),
        pltpu.VMEM((2, EMB, _CHUNK), jnp.float32),
        pltpu.VMEM((2, 16, EMB), jnp.float32),
        pltpu.SemaphoreType.DMA,
        pltpu.SemaphoreType.DMA,
        pltpu.SemaphoreType.DMA,
        pltpu.SemaphoreType.DMA,
    ],
    compiler_params=pltpu.CompilerParams(needs_layout_passes=False),
)
def _gather_sc(table_t_hbm, idx_hbm, out_hbm, idx_v, mitem, mpos, cbuf,
               rstg, sem_c0, sem_c1, sem_w0, sem_w1):
    wid = lax.axis_index("s") * _NC + lax.axis_index("c")
    iota16 = lax.iota(jnp.int32, 16)

    def _prime():
        for t in range(4):
            pltpu.make_async_copy(
                table_t_hbm.at[:, pl.ds(wid * _CHUNK + t * 128, 128)],
                cbuf.at[0, :, pl.ds(t * 128, 128)],
                sem_c0,
            ).start()

    _prime()

    pltpu.sync_copy(idx_hbm, idx_v.at[pl.ds(0, BATCH)])

    def _scan(g, off):
        vec = idx_v[pl.ds(g * 16, 16)]
        m = ((vec >> 9) & 31) == wid
        cnt = plsc.all_reduce_population_count(m)[0]
        plsc.store_compressed(mitem.at[pl.ds(off, 16)], vec, mask=m)
        plsc.store_compressed(
            mpos.at[pl.ds(off, 16)], iota16 + g * 16, mask=m
        )
        return off + cnt

    n_match = lax.fori_loop(0, BATCH // 16, _scan, 0)
    mitem[pl.ds(n_match, 16)] = jnp.full((16,), 1 << 29, jnp.int32)
    ngr = (n_match + 15) >> 4

    def _stream_op(k, is_start):
        b = k & 1
        c = wid + 32 * k

        def _with(sem, buf):
            if is_start:
                for t in range(4):
                    pltpu.make_async_copy(
                        table_t_hbm.at[:, pl.ds(c * _CHUNK + t * 128, 128)],
                        cbuf.at[buf, :, pl.ds(t * 128, 128)],
                        sem,
                    ).start()
            else:
                pltpu.make_async_copy(
                    table_t_hbm.at[:, pl.ds(0, _CHUNK)], cbuf.at[buf], sem
                ).wait()

        lax.cond(
            b == 0,
            lambda: _with(sem_c0, 0),
            lambda: _with(sem_c1, 1),
        )


    def _per_chunk(k, carry):
        c = wid + 32 * k

        def _process():
            b = k & 1

            def _prefetch():
                _stream_op(k + 1, True)

            lax.cond(wid + 32 * (k + 1) < _NFULL, _prefetch, lambda: None)
            _stream_op(k, False)

            def _cscan(m, off2):
                items = mitem[pl.ds(m * 16, 16)]
                cm = (items >> 9) == c
                plsc.store_compressed(
                    idx_v.at[pl.ds(off2, 16)], iota16 + m * 16, mask=cm
                )
                return off2 + plsc.all_reduce_population_count(cm)[0]

            cnt = lax.fori_loop(0, ngr, _cscan, 0)
            idx_v[pl.ds(cnt, 16)] = jnp.full((16,), n_match, jnp.int32)

            def _drain_stage0():
                pltpu.make_async_copy(
                    out_hbm.at[pl.ds(0, 16)], rstg.at[0], sem_w0
                ).wait()

            def _drain_stage1():
                pltpu.make_async_copy(
                    out_hbm.at[pl.ds(0, 16)], rstg.at[1], sem_w1
                ).wait()

            def _drain_stage(sb):
                lax.cond(sb == 0, _drain_stage0, _drain_stage1)

            def _ext(m, carry2):
                refs = idx_v[pl.ds(m * 16, 16)]
                items = plsc.load_gather(mitem, [refs])
                poss = plsc.load_gather(mpos, [refs])
                cm = (items >> 9) == c
                lanes = items & (_CHUNK - 1)
                pos_eff = jnp.where(cm, poss, _DUMP + wid * 16 + iota16)
                sb = m & 1
                lax.cond(m >= 2, lambda: _drain_stage(sb), lambda: None)

                def _scatter_to(stage, sem):
                    for r in range(EMB):
                        v = plsc.load_gather(
                            cbuf.at[b],
                            [jnp.full((16,), r, jnp.int32), lanes],
                            mask=cm,
                        )
                        plsc.store_scatter(
                            stage,
                            [iota16, jnp.full((16,), r, jnp.int32)],
                            v,
                            mask=cm,
                        )
                    for j in range(16):
                        pj = pos_eff[j]
                        pltpu.make_async_copy(
                            stage.at[pl.ds(j, 1)],
                            out_hbm.at[pl.ds(pj, 1)],
                            sem,
                        ).start()

                lax.cond(
                    sb == 0,
                    lambda: _scatter_to(rstg.at[0], sem_w0),
                    lambda: _scatter_to(rstg.at[1], sem_w1),
                )
                return carry2

            ngr2 = (cnt + 15) >> 4
            lax.fori_loop(0, ngr2, _ext, 0)
            lax.cond(ngr2 >= 2, lambda: _drain_stage(ngr2 & 1), lambda: None)
            lax.cond(
                ngr2 >= 1, lambda: _drain_stage((ngr2 - 1) & 1), lambda: None
            )

        lax.cond(c < _NFULL, _process, lambda: None)
        return carry

    lax.fori_loop(0, _K, _per_chunk, 0)


_BS = 2048


def _mlp_body(x_ref, idx_ref, tail_ref, w1_ref, b1_ref, w2_ref, b2_ref,
              o_ref):
    x = x_ref[...]
    t = idx_ref[...] - _TAIL_BASE
    oh = (t == lax.broadcasted_iota(jnp.int32, (_BS, _TAIL), 1)).astype(
        jnp.float32
    )
    tail_x = jnp.dot(oh, tail_ref[...], preferred_element_type=jnp.float32)
    x = jnp.where(t >= 0, tail_x, x)
    h = jnp.dot(x, w1_ref[...], preferred_element_type=jnp.float32)
    h = jnp.maximum(h + b1_ref[...], 0.0)
    o = jnp.dot(h, w2_ref[...], preferred_element_type=jnp.float32)
    o_ref[...] = o + b2_ref[...]


_mlp = pl.pallas_call(
    _mlp_body,
    grid=(BATCH // _BS,),
    in_specs=[
        pl.BlockSpec((_BS, EMB), lambda i: (i, 0)),
        pl.BlockSpec((_BS, 1), lambda i: (i, 0)),
        pl.BlockSpec((_TAIL, EMB), lambda i: (0, 0)),
        pl.BlockSpec((EMB, HID), lambda i: (0, 0)),
        pl.BlockSpec((1, HID), lambda i: (0, 0)),
        pl.BlockSpec((HID, EMB), lambda i: (0, 0)),
        pl.BlockSpec((1, EMB), lambda i: (0, 0)),
    ],
    out_specs=pl.BlockSpec((_BS, EMB), lambda i: (i, 0)),
    out_shape=jax.ShapeDtypeStruct((BATCH, EMB), jnp.float32),
)


def kernel(item_id, item_emb_table, W1, b1, W2, b2):
    idx = item_id.astype(jnp.int32)
    emb_big = _gather_sc(item_emb_table.T, idx)
    tail_rows = item_emb_table[_TAIL_BASE:]
    return _mlp(
        emb_big,
        idx.reshape(BATCH, 1),
        tail_rows,
        W1,
        b1.reshape(1, HID),
        W2,
        b2.reshape(1, EMB),
    )


# speedup vs baseline: 1.0258x; 1.0258x over previous
import functools

import jax
import jax.numpy as jnp
from jax import lax
from jax.experimental import pallas as pl
from jax.experimental.pallas import tpu as pltpu
from jax.experimental.pallas import tpu_sc as plsc

ITEM_NUM = 1000000
EMB = 64
HID = 128
BATCH = 16384

_NC = 2
_NS = 16
_NW = _NC * _NS

_CHUNK = 512
_NFULL = 1953
_TAIL_BASE = _NFULL * _CHUNK
_TAIL = ITEM_NUM - _TAIL_BASE
_K = 62
_DUMP = BATCH

_sc_mesh = plsc.VectorSubcoreMesh(core_axis_name="c", subcore_axis_name="s")


@functools.partial(
    pl.kernel,
    mesh=_sc_mesh,
    out_type=jax.ShapeDtypeStruct((BATCH + 16 * _NW, EMB), jnp.float32),
    scratch_types=[
        pltpu.VMEM((BATCH + 16,), jnp.int32),
        pltpu.VMEM((BATCH + 16,), jnp.int32),
        pltpu.VMEM((BATCH + 16,), jnp.int32),
        pltpu.VMEM((2, EMB, _CHUNK), jnp.float32),
        pltpu.VMEM((2, 16, EMB), jnp.float32),
        pltpu.SemaphoreType.DMA,
        pltpu.SemaphoreType.DMA,
        pltpu.SemaphoreType.DMA,
        pltpu.SemaphoreType.DMA,
    ],
    compiler_params=pltpu.CompilerParams(needs_layout_passes=False),
)
def _gather_sc(table_t_hbm, idx_hbm, out_hbm, idx_v, mitem, mpos, cbuf,
               rstg, sem_c0, sem_c1, sem_w0, sem_w1):
    wid = lax.axis_index("s") * _NC + lax.axis_index("c")
    iota16 = lax.iota(jnp.int32, 16)

    def _prime():
        for buf, sem in ((0, sem_c0), (1, sem_c1)):
            base = (wid + 32 * buf) * _CHUNK
            for t in range(4):
                pltpu.make_async_copy(
                    table_t_hbm.at[:, pl.ds(base + t * 128, 128)],
                    cbuf.at[buf, :, pl.ds(t * 128, 128)],
                    sem,
                ).start()

    _prime()

    pltpu.sync_copy(idx_hbm, idx_v.at[pl.ds(0, BATCH)])

    def _scan(g, off):
        vec = idx_v[pl.ds(g * 16, 16)]
        m = ((vec >> 9) & 31) == wid
        cnt = plsc.all_reduce_population_count(m)[0]
        plsc.store_compressed(mitem.at[pl.ds(off, 16)], vec, mask=m)
        plsc.store_compressed(
            mpos.at[pl.ds(off, 16)], iota16 + g * 16, mask=m
        )
        return off + cnt

    n_match = lax.fori_loop(0, BATCH // 16, _scan, 0)
    mitem[pl.ds(n_match, 16)] = jnp.full((16,), 1 << 29, jnp.int32)
    ngr = (n_match + 15) >> 4

    def _stream_op(k, is_start):
        b = k & 1
        c = wid + 32 * k

        def _with(sem, buf):
            if is_start:
                for t in range(4):
                    pltpu.make_async_copy(
                        table_t_hbm.at[:, pl.ds(c * _CHUNK + t * 128, 128)],
                        cbuf.at[buf, :, pl.ds(t * 128, 128)],
                        sem,
                    ).start()
            else:
                pltpu.make_async_copy(
                    table_t_hbm.at[:, pl.ds(0, _CHUNK)], cbuf.at[buf], sem
                ).wait()

        lax.cond(
            b == 0,
            lambda: _with(sem_c0, 0),
            lambda: _with(sem_c1, 1),
        )


    def _per_chunk(k, carry):
        c = wid + 32 * k

        def _process():
            b = k & 1

            def _prefetch():
                _stream_op(k + 1, True)

            lax.cond(
                jnp.logical_and(k >= 1, wid + 32 * (k + 1) < _NFULL),
                _prefetch,
                lambda: None,
            )
            _stream_op(k, False)

            def _cscan(m, off2):
                items = mitem[pl.ds(m * 16, 16)]
                cm = (items >> 9) == c
                plsc.store_compressed(
                    idx_v.at[pl.ds(off2, 16)], iota16 + m * 16, mask=cm
                )
                return off2 + plsc.all_reduce_population_count(cm)[0]

            cnt = lax.fori_loop(0, ngr, _cscan, 0)
            idx_v[pl.ds(cnt, 16)] = jnp.full((16,), n_match, jnp.int32)

            def _drain_stage0():
                pltpu.make_async_copy(
                    out_hbm.at[pl.ds(0, 16)], rstg.at[0], sem_w0
                ).wait()

            def _drain_stage1():
                pltpu.make_async_copy(
                    out_hbm.at[pl.ds(0, 16)], rstg.at[1], sem_w1
                ).wait()

            def _drain_stage(sb):
                lax.cond(sb == 0, _drain_stage0, _drain_stage1)

            def _ext(m, carry2):
                refs = idx_v[pl.ds(m * 16, 16)]
                items = plsc.load_gather(mitem, [refs])
                poss = plsc.load_gather(mpos, [refs])
                cm = (items >> 9) == c
                lanes = items & (_CHUNK - 1)
                pos_eff = jnp.where(cm, poss, _DUMP + wid * 16 + iota16)
                sb = m & 1
                lax.cond(m >= 2, lambda: _drain_stage(sb), lambda: None)

                def _scatter_to(stage, sem):
                    for r in range(EMB):
                        v = plsc.load_gather(
                            cbuf.at[b],
                            [jnp.full((16,), r, jnp.int32), lanes],
                            mask=cm,
                        )
                        plsc.store_scatter(
                            stage,
                            [iota16, jnp.full((16,), r, jnp.int32)],
                            v,
                            mask=cm,
                        )
                    for j in range(16):
                        pj = pos_eff[j]
                        pltpu.make_async_copy(
                            stage.at[pl.ds(j, 1)],
                            out_hbm.at[pl.ds(pj, 1)],
                            sem,
                        ).start()

                lax.cond(
                    sb == 0,
                    lambda: _scatter_to(rstg.at[0], sem_w0),
                    lambda: _scatter_to(rstg.at[1], sem_w1),
                )
                return carry2

            ngr2 = (cnt + 15) >> 4
            lax.fori_loop(0, ngr2, _ext, 0)
            lax.cond(ngr2 >= 2, lambda: _drain_stage(ngr2 & 1), lambda: None)
            lax.cond(
                ngr2 >= 1, lambda: _drain_stage((ngr2 - 1) & 1), lambda: None
            )

        lax.cond(c < _NFULL, _process, lambda: None)
        return carry

    lax.fori_loop(0, _K, _per_chunk, 0)


_BS = 2048


def _mlp_body(x_ref, idx_ref, tail_ref, w1_ref, b1_ref, w2_ref, b2_ref,
              o_ref):
    x = x_ref[...]
    t = idx_ref[...] - _TAIL_BASE
    oh = (t == lax.broadcasted_iota(jnp.int32, (_BS, _TAIL), 1)).astype(
        jnp.float32
    )
    tail_x = jnp.dot(oh, tail_ref[...], preferred_element_type=jnp.float32)
    x = jnp.where(t >= 0, tail_x, x)
    h = jnp.dot(x, w1_ref[...], preferred_element_type=jnp.float32)
    h = jnp.maximum(h + b1_ref[...], 0.0)
    ot = jax.lax.dot_general(
        w2_ref[...], h, (((0,), (1,)), ((), ())),
        preferred_element_type=jnp.float32,
    )
    o_ref[...] = ot + b2_ref[...]


_mlp = pl.pallas_call(
    _mlp_body,
    grid=(BATCH // _BS,),
    in_specs=[
        pl.BlockSpec((_BS, EMB), lambda i: (i, 0)),
        pl.BlockSpec((_BS, 1), lambda i: (i, 0)),
        pl.BlockSpec((_TAIL, EMB), lambda i: (0, 0)),
        pl.BlockSpec((EMB, HID), lambda i: (0, 0)),
        pl.BlockSpec((1, HID), lambda i: (0, 0)),
        pl.BlockSpec((HID, EMB), lambda i: (0, 0)),
        pl.BlockSpec((EMB, 1), lambda i: (0, 0)),
    ],
    out_specs=pl.BlockSpec((EMB, _BS), lambda i: (0, i)),
    out_shape=jax.ShapeDtypeStruct((EMB, BATCH), jnp.float32),
)


def kernel(item_id, item_emb_table, W1, b1, W2, b2):
    idx = item_id.astype(jnp.int32)
    emb_big = _gather_sc(item_emb_table.T, idx)
    tail_rows = item_emb_table[_TAIL_BASE:]
    out_t = _mlp(
        emb_big,
        idx.reshape(BATCH, 1),
        tail_rows,
        W1,
        b1.reshape(1, HID),
        W2,
        b2.reshape(EMB, 1),
    )
    return out_t.T


# speedup vs baseline: 1.0472x; 1.0208x over previous
import functools

import jax
import jax.numpy as jnp
from jax import lax
from jax.experimental import pallas as pl
from jax.experimental.pallas import tpu as pltpu
from jax.experimental.pallas import tpu_sc as plsc

ITEM_NUM = 1000000
EMB = 64
HID = 128
BATCH = 16384

_NC = 2
_NS = 16
_NW = _NC * _NS

_CHUNK = 512
_NFULL = 1953
_TAIL_BASE = _NFULL * _CHUNK
_TAIL = ITEM_NUM - _TAIL_BASE
_K = 62
_DUMP = BATCH

_sc_mesh = plsc.VectorSubcoreMesh(core_axis_name="c", subcore_axis_name="s")


@functools.partial(
    pl.kernel,
    mesh=_sc_mesh,
    out_type=jax.ShapeDtypeStruct((BATCH + 16 * _NW, EMB), jnp.float32),
    scratch_types=[
        pltpu.VMEM((BATCH + 16,), jnp.int32),
        pltpu.VMEM((BATCH + 16,), jnp.int32),
        pltpu.VMEM((BATCH + 16,), jnp.int32),
        pltpu.VMEM((2, EMB, _CHUNK), jnp.float32),
        pltpu.VMEM((2, 16, EMB), jnp.float32),
        pltpu.SemaphoreType.DMA,
        pltpu.SemaphoreType.DMA,
        pltpu.SemaphoreType.DMA,
        pltpu.SemaphoreType.DMA,
    ],
    compiler_params=pltpu.CompilerParams(needs_layout_passes=False),
)
def _gather_sc(table_t_hbm, idx_hbm, out_hbm, idx_v, mitem, mpos, cbuf,
               rstg, sem_c0, sem_c1, sem_w0, sem_w1):
    wid = lax.axis_index("s") * _NC + lax.axis_index("c")
    iota16 = lax.iota(jnp.int32, 16)

    def _prime():
        for buf, sem in ((0, sem_c0), (1, sem_c1)):
            base = (wid + 32 * buf) * _CHUNK
            for t in range(4):
                pltpu.make_async_copy(
                    table_t_hbm.at[:, pl.ds(base + t * 128, 128)],
                    cbuf.at[buf, :, pl.ds(t * 128, 128)],
                    sem,
                ).start()

    _prime()

    pltpu.sync_copy(idx_hbm, idx_v.at[pl.ds(0, BATCH)])

    def _scan(g, off):
        vec = idx_v[pl.ds(g * 16, 16)]
        m = ((vec >> 9) & 31) == wid
        cnt = plsc.all_reduce_population_count(m)[0]
        plsc.store_compressed(mitem.at[pl.ds(off, 16)], vec, mask=m)
        plsc.store_compressed(
            mpos.at[pl.ds(off, 16)], iota16 + g * 16, mask=m
        )
        return off + cnt

    n_match = lax.fori_loop(0, BATCH // 16, _scan, 0)
    mitem[pl.ds(n_match, 16)] = jnp.full((16,), 1 << 29, jnp.int32)
    ngr = (n_match + 15) >> 4

    def _stream_op(k, is_start):
        b = k & 1
        c = wid + 32 * k

        def _with(sem, buf):
            if is_start:
                for t in range(4):
                    pltpu.make_async_copy(
                        table_t_hbm.at[:, pl.ds(c * _CHUNK + t * 128, 128)],
                        cbuf.at[buf, :, pl.ds(t * 128, 128)],
                        sem,
                    ).start()
            else:
                pltpu.make_async_copy(
                    table_t_hbm.at[:, pl.ds(0, _CHUNK)], cbuf.at[buf], sem
                ).wait()

        lax.cond(
            b == 0,
            lambda: _with(sem_c0, 0),
            lambda: _with(sem_c1, 1),
        )


    def _per_chunk(k, carry):
        c = wid + 32 * k

        def _process():
            b = k & 1

            def _prefetch():
                _stream_op(k + 1, True)

            lax.cond(
                jnp.logical_and(k >= 1, wid + 32 * (k + 1) < _NFULL),
                _prefetch,
                lambda: None,
            )
            _stream_op(k, False)

            def _cscan(m, off2):
                items = mitem[pl.ds(m * 16, 16)]
                cm = (items >> 9) == c
                plsc.store_compressed(
                    idx_v.at[pl.ds(off2, 16)], iota16 + m * 16, mask=cm
                )
                return off2 + plsc.all_reduce_population_count(cm)[0]

            cnt = lax.fori_loop(0, ngr, _cscan, 0)
            idx_v[pl.ds(cnt, 16)] = jnp.full((16,), n_match, jnp.int32)

            def _drain_stage0():
                pltpu.make_async_copy(
                    out_hbm.at[pl.ds(0, 16)], rstg.at[0], sem_w0
                ).wait()

            def _drain_stage1():
                pltpu.make_async_copy(
                    out_hbm.at[pl.ds(0, 16)], rstg.at[1], sem_w1
                ).wait()

            def _drain_stage(sb):
                lax.cond(sb == 0, _drain_stage0, _drain_stage1)

            def _ext(m, carry2):
                refs = idx_v[pl.ds(m * 16, 16)]
                items = plsc.load_gather(mitem, [refs])
                poss = plsc.load_gather(mpos, [refs])
                cm = (items >> 9) == c
                lanes = items & (_CHUNK - 1)
                pos_eff = jnp.where(cm, poss, _DUMP + wid * 16 + iota16)
                sb = m & 1
                lax.cond(m >= 2, lambda: _drain_stage(sb), lambda: None)

                def _scatter_to(stage, sem):
                    for r in range(EMB):
                        v = plsc.load_gather(
                            cbuf.at[b],
                            [jnp.full((16,), r, jnp.int32), lanes],
                            mask=cm,
                        )
                        plsc.store_scatter(
                            stage,
                            [iota16, jnp.full((16,), r, jnp.int32)],
                            v,
                            mask=cm,
                        )
                    for j in range(16):
                        pj = pos_eff[j]
                        pltpu.make_async_copy(
                            stage.at[pl.ds(j, 1)],
                            out_hbm.at[pl.ds(pj, 1)],
                            sem,
                        ).start()

                lax.cond(
                    sb == 0,
                    lambda: _scatter_to(rstg.at[0], sem_w0),
                    lambda: _scatter_to(rstg.at[1], sem_w1),
                )
                return carry2

            ngr2 = (cnt + 15) >> 4
            lax.fori_loop(0, ngr2, _ext, 0)
            lax.cond(ngr2 >= 2, lambda: _drain_stage(ngr2 & 1), lambda: None)
            lax.cond(
                ngr2 >= 1, lambda: _drain_stage((ngr2 - 1) & 1), lambda: None
            )

        lax.cond(c < _NFULL, _process, lambda: None)
        return carry

    lax.fori_loop(0, _K, _per_chunk, 0)


_BS = 4096


def _mlp_body(x_ref, idx_ref, tail_ref, w1_ref, b1_ref, w2_ref, b2_ref,
              o_ref):
    x = x_ref[...]
    t = idx_ref[...] - _TAIL_BASE
    oh = (t == lax.broadcasted_iota(jnp.int32, (_BS, _TAIL), 1)).astype(
        jnp.float32
    )
    tail_x = jnp.dot(oh, tail_ref[...], preferred_element_type=jnp.float32)
    x = jnp.where(t >= 0, tail_x, x)
    h = jnp.dot(x, w1_ref[...], preferred_element_type=jnp.float32)
    h = jnp.maximum(h + b1_ref[...], 0.0)
    ot = jax.lax.dot_general(
        w2_ref[...], h, (((0,), (1,)), ((), ())),
        preferred_element_type=jnp.float32,
    )
    o_ref[...] = ot + b2_ref[...]


_mlp = pl.pallas_call(
    _mlp_body,
    grid=(BATCH // _BS,),
    in_specs=[
        pl.BlockSpec((_BS, EMB), lambda i: (i, 0)),
        pl.BlockSpec((_BS, 1), lambda i: (i, 0)),
        pl.BlockSpec((_TAIL, EMB), lambda i: (0, 0)),
        pl.BlockSpec((EMB, HID), lambda i: (0, 0)),
        pl.BlockSpec((1, HID), lambda i: (0, 0)),
        pl.BlockSpec((HID, EMB), lambda i: (0, 0)),
        pl.BlockSpec((EMB, 1), lambda i: (0, 0)),
    ],
    out_specs=pl.BlockSpec((EMB, _BS), lambda i: (0, i)),
    out_shape=jax.ShapeDtypeStruct((EMB, BATCH), jnp.float32),
)


def kernel(item_id, item_emb_table, W1, b1, W2, b2):
    idx = item_id.astype(jnp.int32)
    emb_big = _gather_sc(item_emb_table.T, idx)
    tail_rows = item_emb_table[_TAIL_BASE:]
    out_t = _mlp(
        emb_big,
        idx.reshape(BATCH, 1),
        tail_rows,
        W1,
        b1.reshape(1, HID),
        W2,
        b2.reshape(EMB, 1),
    )
    return out_t.T


# speedup vs baseline: 1.0477x; 1.0006x over previous
import functools

import jax
import jax.numpy as jnp
from jax import lax
from jax.experimental import pallas as pl
from jax.experimental.pallas import tpu as pltpu
from jax.experimental.pallas import tpu_sc as plsc

ITEM_NUM = 1000000
EMB = 64
HID = 128
BATCH = 16384

_NC = 2
_NS = 16
_NW = _NC * _NS

_CHUNK = 512
_NFULL = 1953
_TAIL_BASE = _NFULL * _CHUNK
_TAIL = ITEM_NUM - _TAIL_BASE
_K = 62
_DUMP = BATCH

_sc_mesh = plsc.VectorSubcoreMesh(core_axis_name="c", subcore_axis_name="s")


@functools.partial(
    pl.kernel,
    mesh=_sc_mesh,
    out_type=jax.ShapeDtypeStruct((BATCH + 16 * _NW, EMB), jnp.float32),
    scratch_types=[
        pltpu.VMEM((BATCH + 16,), jnp.int32),
        pltpu.VMEM((BATCH + 16,), jnp.int32),
        pltpu.VMEM((BATCH + 16,), jnp.int32),
        pltpu.VMEM((2, EMB, _CHUNK), jnp.float32),
        pltpu.VMEM((2, 16, EMB), jnp.float32),
        pltpu.SemaphoreType.DMA,
        pltpu.SemaphoreType.DMA,
        pltpu.SemaphoreType.DMA,
        pltpu.SemaphoreType.DMA,
    ],
    compiler_params=pltpu.CompilerParams(needs_layout_passes=False),
)
def _gather_sc(table_t_hbm, idx_hbm, out_hbm, idx_v, mitem, mpos, cbuf,
               rstg, sem_c0, sem_c1, sem_w0, sem_w1):
    wid = lax.axis_index("s") * _NC + lax.axis_index("c")
    iota16 = lax.iota(jnp.int32, 16)

    def _prime():
        for buf, sem in ((0, sem_c0), (1, sem_c1)):
            base = (wid + 32 * buf) * _CHUNK
            for t in range(8):
                pltpu.make_async_copy(
                    table_t_hbm.at[pl.ds(t * 8, 8), pl.ds(base, _CHUNK)],
                    cbuf.at[buf, pl.ds(t * 8, 8)],
                    sem,
                ).start()

    _prime()

    pltpu.sync_copy(idx_hbm, idx_v.at[pl.ds(0, BATCH)])

    def _scan(g, off):
        vec = idx_v[pl.ds(g * 16, 16)]
        m = ((vec >> 9) & 31) == wid
        cnt = plsc.all_reduce_population_count(m)[0]
        plsc.store_compressed(mitem.at[pl.ds(off, 16)], vec, mask=m)
        plsc.store_compressed(
            mpos.at[pl.ds(off, 16)], iota16 + g * 16, mask=m
        )
        return off + cnt

    n_match = lax.fori_loop(0, BATCH // 16, _scan, 0)
    mitem[pl.ds(n_match, 16)] = jnp.full((16,), 1 << 29, jnp.int32)
    ngr = (n_match + 15) >> 4

    def _stream_op(k, is_start):
        b = k & 1
        c = wid + 32 * k

        def _with(sem, buf):
            if is_start:
                for t in range(8):
                    pltpu.make_async_copy(
                        table_t_hbm.at[pl.ds(t * 8, 8), pl.ds(c * _CHUNK, _CHUNK)],
                        cbuf.at[buf, pl.ds(t * 8, 8)],
                        sem,
                    ).start()
            else:
                pltpu.make_async_copy(
                    table_t_hbm.at[:, pl.ds(0, _CHUNK)], cbuf.at[buf], sem
                ).wait()

        lax.cond(
            b == 0,
            lambda: _with(sem_c0, 0),
            lambda: _with(sem_c1, 1),
        )


    def _per_chunk(k, carry):
        c = wid + 32 * k

        def _process():
            b = k & 1

            def _prefetch():
                _stream_op(k + 1, True)

            lax.cond(
                jnp.logical_and(k >= 1, wid + 32 * (k + 1) < _NFULL),
                _prefetch,
                lambda: None,
            )
            _stream_op(k, False)

            def _cscan(m, off2):
                items = mitem[pl.ds(m * 16, 16)]
                cm = (items >> 9) == c
                plsc.store_compressed(
                    idx_v.at[pl.ds(off2, 16)], iota16 + m * 16, mask=cm
                )
                return off2 + plsc.all_reduce_population_count(cm)[0]

            cnt = lax.fori_loop(0, ngr, _cscan, 0)
            idx_v[pl.ds(cnt, 16)] = jnp.full((16,), n_match, jnp.int32)

            def _drain_stage0():
                pltpu.make_async_copy(
                    out_hbm.at[pl.ds(0, 16)], rstg.at[0], sem_w0
                ).wait()

            def _drain_stage1():
                pltpu.make_async_copy(
                    out_hbm.at[pl.ds(0, 16)], rstg.at[1], sem_w1
                ).wait()

            def _drain_stage(sb):
                lax.cond(sb == 0, _drain_stage0, _drain_stage1)

            def _ext(m, carry2):
                refs = idx_v[pl.ds(m * 16, 16)]
                items = plsc.load_gather(mitem, [refs])
                poss = plsc.load_gather(mpos, [refs])
                cm = (items >> 9) == c
                lanes = items & (_CHUNK - 1)
                pos_eff = jnp.where(cm, poss, _DUMP + wid * 16 + iota16)
                sb = m & 1
                lax.cond(m >= 2, lambda: _drain_stage(sb), lambda: None)

                def _scatter_to(stage, sem):
                    for r in range(EMB):
                        v = plsc.load_gather(
                            cbuf.at[b],
                            [jnp.full((16,), r, jnp.int32), lanes],
                            mask=cm,
                        )
                        plsc.store_scatter(
                            stage,
                            [iota16, jnp.full((16,), r, jnp.int32)],
                            v,
                            mask=cm,
                        )
                    for j in range(16):
                        pj = pos_eff[j]
                        pltpu.make_async_copy(
                            stage.at[pl.ds(j, 1)],
                            out_hbm.at[pl.ds(pj, 1)],
                            sem,
                        ).start()

                lax.cond(
                    sb == 0,
                    lambda: _scatter_to(rstg.at[0], sem_w0),
                    lambda: _scatter_to(rstg.at[1], sem_w1),
                )
                return carry2

            ngr2 = (cnt + 15) >> 4
            lax.fori_loop(0, ngr2, _ext, 0)
            lax.cond(ngr2 >= 2, lambda: _drain_stage(ngr2 & 1), lambda: None)
            lax.cond(
                ngr2 >= 1, lambda: _drain_stage((ngr2 - 1) & 1), lambda: None
            )

        lax.cond(c < _NFULL, _process, lambda: None)
        return carry

    lax.fori_loop(0, _K, _per_chunk, 0)


_BS = 8192


def _mlp_body(x_ref, idx_ref, tail_ref, w1_ref, b1_ref, w2_ref, b2_ref,
              o_ref):
    x = x_ref[...]
    t = idx_ref[...] - _TAIL_BASE
    oh = (t == lax.broadcasted_iota(jnp.int32, (_BS, _TAIL), 1)).astype(
        jnp.float32
    )
    tail_x = jnp.dot(oh, tail_ref[...], preferred_element_type=jnp.float32)
    x = jnp.where(t >= 0, tail_x, x)
    h = jnp.dot(x, w1_ref[...], preferred_element_type=jnp.float32)
    h = jnp.maximum(h + b1_ref[...], 0.0)
    ot = jax.lax.dot_general(
        w2_ref[...], h, (((0,), (1,)), ((), ())),
        preferred_element_type=jnp.float32,
    )
    o_ref[...] = ot + b2_ref[...]


_mlp = pl.pallas_call(
    _mlp_body,
    grid=(BATCH // _BS,),
    in_specs=[
        pl.BlockSpec((_BS, EMB), lambda i: (i, 0)),
        pl.BlockSpec((_BS, 1), lambda i: (i, 0)),
        pl.BlockSpec((_TAIL, EMB), lambda i: (0, 0)),
        pl.BlockSpec((EMB, HID), lambda i: (0, 0)),
        pl.BlockSpec((1, HID), lambda i: (0, 0)),
        pl.BlockSpec((HID, EMB), lambda i: (0, 0)),
        pl.BlockSpec((EMB, 1), lambda i: (0, 0)),
    ],
    out_specs=pl.BlockSpec((EMB, _BS), lambda i: (0, i)),
    out_shape=jax.ShapeDtypeStruct((EMB, BATCH), jnp.float32),
)


def kernel(item_id, item_emb_table, W1, b1, W2, b2):
    idx = item_id.astype(jnp.int32)
    emb_big = _gather_sc(item_emb_table.T, idx)
    tail_rows = item_emb_table[_TAIL_BASE:]
    out_t = _mlp(
        emb_big,
        idx.reshape(BATCH, 1),
        tail_rows,
        W1,
        b1.reshape(1, HID),
        W2,
        b2.reshape(EMB, 1),
    )
    return out_t.T


# speedup vs baseline: 1.0718x; 1.0230x over previous
import functools

import jax
import jax.numpy as jnp
from jax import lax
from jax.experimental import pallas as pl
from jax.experimental.pallas import tpu as pltpu
from jax.experimental.pallas import tpu_sc as plsc

ITEM_NUM = 1000000
EMB = 64
HID = 128
BATCH = 16384

_NC = 2
_NS = 16
_NW = _NC * _NS

_CHUNK = 512
_NFULL = 1953
_TAIL_BASE = _NFULL * _CHUNK
_TAIL = ITEM_NUM - _TAIL_BASE
_K = 62
_DUMP = BATCH

_sc_mesh = plsc.VectorSubcoreMesh(core_axis_name="c", subcore_axis_name="s")


@functools.partial(
    pl.kernel,
    mesh=_sc_mesh,
    out_type=jax.ShapeDtypeStruct((BATCH + 16 * _NW, EMB), jnp.float32),
    scratch_types=[
        pltpu.VMEM((BATCH + 16,), jnp.int32),
        pltpu.VMEM((BATCH + 16,), jnp.int32),
        pltpu.VMEM((BATCH + 16,), jnp.int32),
        pltpu.VMEM((2, EMB, _CHUNK), jnp.float32),
        pltpu.VMEM((2, 16, EMB), jnp.float32),
        pltpu.VMEM((_TAIL, EMB), jnp.float32),
        pltpu.SemaphoreType.DMA,
        pltpu.SemaphoreType.DMA,
        pltpu.SemaphoreType.DMA,
        pltpu.SemaphoreType.DMA,
    ],
    compiler_params=pltpu.CompilerParams(needs_layout_passes=False),
)
def _gather_sc(table_t_hbm, idx_hbm, tail_hbm, out_hbm, idx_v, mitem, mpos,
               cbuf, rstg, tbuf, sem_c0, sem_c1, sem_w0, sem_w1):
    wid = lax.axis_index("s") * _NC + lax.axis_index("c")
    iota16 = lax.iota(jnp.int32, 16)

    def _prime():
        for buf, sem in ((0, sem_c0), (1, sem_c1)):
            base = (wid + 32 * buf) * _CHUNK
            for t in range(8):
                pltpu.make_async_copy(
                    table_t_hbm.at[pl.ds(t * 8, 8), pl.ds(base, _CHUNK)],
                    cbuf.at[buf, pl.ds(t * 8, 8)],
                    sem,
                ).start()

    _prime()

    pltpu.sync_copy(idx_hbm, idx_v.at[pl.ds(0, BATCH)])
    pltpu.sync_copy(tail_hbm, tbuf)

    def _scan(g, off):
        vec = idx_v[pl.ds(g * 16, 16)]
        m = ((vec >> 9) & 31) == wid
        cnt = plsc.all_reduce_population_count(m)[0]
        plsc.store_compressed(mitem.at[pl.ds(off, 16)], vec, mask=m)
        plsc.store_compressed(
            mpos.at[pl.ds(off, 16)], iota16 + g * 16, mask=m
        )
        return off + cnt

    n_match = lax.fori_loop(0, BATCH // 16, _scan, 0)
    mitem[pl.ds(n_match, 16)] = jnp.full((16,), 1 << 29, jnp.int32)
    ngr = (n_match + 15) >> 4

    def _stream_op(k, is_start):
        b = k & 1
        c = wid + 32 * k

        def _with(sem, buf):
            if is_start:
                for t in range(8):
                    pltpu.make_async_copy(
                        table_t_hbm.at[pl.ds(t * 8, 8), pl.ds(c * _CHUNK, _CHUNK)],
                        cbuf.at[buf, pl.ds(t * 8, 8)],
                        sem,
                    ).start()
            else:
                pltpu.make_async_copy(
                    table_t_hbm.at[:, pl.ds(0, _CHUNK)], cbuf.at[buf], sem
                ).wait()

        lax.cond(
            b == 0,
            lambda: _with(sem_c0, 0),
            lambda: _with(sem_c1, 1),
        )


    def _per_chunk(k, carry):
        c = wid + 32 * k

        def _process():
            b = k & 1

            def _prefetch():
                _stream_op(k + 1, True)

            lax.cond(
                jnp.logical_and(k >= 1, wid + 32 * (k + 1) < _NFULL),
                _prefetch,
                lambda: None,
            )
            lax.cond(
                c < _NFULL, lambda: _stream_op(k, False), lambda: None
            )

            def _cscan(m, off2):
                items = mitem[pl.ds(m * 16, 16)]
                cm = (items >> 9) == c
                plsc.store_compressed(
                    idx_v.at[pl.ds(off2, 16)], iota16 + m * 16, mask=cm
                )
                return off2 + plsc.all_reduce_population_count(cm)[0]

            cnt = lax.fori_loop(0, ngr, _cscan, 0)
            idx_v[pl.ds(cnt, 16)] = jnp.full((16,), n_match, jnp.int32)

            def _drain_stage0():
                pltpu.make_async_copy(
                    out_hbm.at[pl.ds(0, 16)], rstg.at[0], sem_w0
                ).wait()

            def _drain_stage1():
                pltpu.make_async_copy(
                    out_hbm.at[pl.ds(0, 16)], rstg.at[1], sem_w1
                ).wait()

            def _drain_stage(sb):
                lax.cond(sb == 0, _drain_stage0, _drain_stage1)

            def _ext(m, carry2):
                refs = idx_v[pl.ds(m * 16, 16)]
                items = plsc.load_gather(mitem, [refs])
                poss = plsc.load_gather(mpos, [refs])
                cm = (items >> 9) == c
                lanes = items & (_CHUNK - 1)
                pos_eff = jnp.where(cm, poss, _DUMP + wid * 16 + iota16)
                sb = m & 1
                lax.cond(m >= 2, lambda: _drain_stage(sb), lambda: None)

                def _scatter_to(stage, sem, from_tail):
                    for r in range(EMB):
                        if from_tail:
                            v = plsc.load_gather(
                                tbuf,
                                [lanes, jnp.full((16,), r, jnp.int32)],
                                mask=cm,
                            )
                        else:
                            v = plsc.load_gather(
                                cbuf.at[b],
                                [jnp.full((16,), r, jnp.int32), lanes],
                                mask=cm,
                            )
                        plsc.store_scatter(
                            stage,
                            [iota16, jnp.full((16,), r, jnp.int32)],
                            v,
                            mask=cm,
                        )
                    for j in range(16):
                        pj = pos_eff[j]
                        pltpu.make_async_copy(
                            stage.at[pl.ds(j, 1)],
                            out_hbm.at[pl.ds(pj, 1)],
                            sem,
                        ).start()

                def _go(stage, sem):
                    lax.cond(
                        c == _NFULL,
                        lambda: _scatter_to(stage, sem, True),
                        lambda: _scatter_to(stage, sem, False),
                    )

                lax.cond(
                    sb == 0,
                    lambda: _go(rstg.at[0], sem_w0),
                    lambda: _go(rstg.at[1], sem_w1),
                )
                return carry2

            ngr2 = (cnt + 15) >> 4
            lax.fori_loop(0, ngr2, _ext, 0)
            lax.cond(ngr2 >= 2, lambda: _drain_stage(ngr2 & 1), lambda: None)
            lax.cond(
                ngr2 >= 1, lambda: _drain_stage((ngr2 - 1) & 1), lambda: None
            )

        lax.cond(c <= _NFULL, _process, lambda: None)
        return carry

    lax.fori_loop(0, _K, _per_chunk, 0)


_BS = 8192


def _mlp_body(x_ref, w1_ref, b1_ref, w2_ref, b2_ref, o_ref):
    x = x_ref[...]
    h = jnp.dot(x, w1_ref[...], preferred_element_type=jnp.float32)
    h = jnp.maximum(h + b1_ref[...], 0.0)
    ot = jax.lax.dot_general(
        w2_ref[...], h, (((0,), (1,)), ((), ())),
        preferred_element_type=jnp.float32,
    )
    o_ref[...] = ot + b2_ref[...]


_mlp = pl.pallas_call(
    _mlp_body,
    grid=(BATCH // _BS,),
    in_specs=[
        pl.BlockSpec((_BS, EMB), lambda i: (i, 0)),
        pl.BlockSpec((EMB, HID), lambda i: (0, 0)),
        pl.BlockSpec((1, HID), lambda i: (0, 0)),
        pl.BlockSpec((HID, EMB), lambda i: (0, 0)),
        pl.BlockSpec((EMB, 1), lambda i: (0, 0)),
    ],
    out_specs=pl.BlockSpec((EMB, _BS), lambda i: (0, i)),
    out_shape=jax.ShapeDtypeStruct((EMB, BATCH), jnp.float32),
)


def kernel(item_id, item_emb_table, W1, b1, W2, b2):
    idx = item_id.astype(jnp.int32)
    tail_rows = item_emb_table[_TAIL_BASE:]
    emb_big = _gather_sc(item_emb_table.T, idx, tail_rows)
    out_t = _mlp(emb_big, W1, b1.reshape(1, HID), W2, b2.reshape(EMB, 1))
    return out_t.T
